# Initial kernel scaffold; baseline (speedup 1.0000x reference)
import jax
import jax.numpy as jnp
from jax.experimental import pallas as pl


def kernel(adj_matrix, embeddings, edge_index, W_user, b_user, W_item, b_item):
    raise NotImplementedError("write your pallas kernel here")


# speedup vs baseline: 2.2339x; 2.2339x over previous
import functools

import jax
import jax.numpy as jnp
from jax import lax
from jax.experimental import pallas as pl
from jax.experimental.pallas import tpu as pltpu
from jax.experimental.pallas import tpu_sc as plsc

N = 10000
E = 320000
D = 128
SEP = N // 2

NC = 2
NS = 16
NW = NC * NS
B = 128
EPW = ((E + NW * B - 1) // (NW * B)) * B
EPAD = EPW * NW
NB = EPW // B

ACC_ROWS = 10240
TPW = ACC_ROWS // NS
ZR = 64


def _build_t_kernel(emb_ref, wu_ref, bu_ref, wi_ref, bi_ref, out_ref):
    nb = pl.num_programs(0) // 2
    is_user = pl.program_id(0) < nb
    mu = lax.dot_general(wu_ref[...], wi_ref[...], (((0,), (0,)), ((), ())))
    mi = lax.dot_general(wi_ref[...], wu_ref[...], (((0,), (0,)), ((), ())))
    cu = jnp.dot(bu_ref[...], wi_ref[...])
    ci = jnp.dot(bi_ref[...], wu_ref[...])
    m = jnp.where(is_user, mu, mi)
    c = jnp.where(is_user, cu, ci)
    out_ref[...] = jnp.dot(emb_ref[...], m, preferred_element_type=jnp.float32) + c


def _build_t(embeddings, w_user, b_user, w_item, b_item):
    bn = 400
    nb = N // bn
    return pl.pallas_call(
        _build_t_kernel,
        grid=(2 * nb,),
        in_specs=[
            pl.BlockSpec((bn, D), lambda g: (g % nb, 0)),
            pl.BlockSpec((D, D), lambda g: (0, 0)),
            pl.BlockSpec((1, D), lambda g: (0, 0)),
            pl.BlockSpec((D, D), lambda g: (0, 0)),
            pl.BlockSpec((1, D), lambda g: (0, 0)),
        ],
        out_specs=pl.BlockSpec((bn, D), lambda g: (g, 0)),
        out_shape=jax.ShapeDtypeStruct((2 * N, D), jnp.float32),
    )(embeddings, w_user, b_user.reshape(1, D), w_item, b_item.reshape(1, D))


def _sc_kernel(adj_hbm, t_hbm, fidx_hbm, tidx_hbm, uidx_hbm, out_hbm,
               fidx_v, tidx_v, uidx_v, w_v, rows_v, zero_v, acc, sem_a, sem_b):
    c = lax.axis_index("c")
    s = lax.axis_index("s")
    wid = s * NC + c

    def zfill(r, _):
        for k in range(D // 16):
            zero_v[r, pl.ds(k * 16, 16)] = jnp.zeros((16,), jnp.float32)
        return 0
    lax.fori_loop(0, ZR, zfill, 0)
    for k in range(TPW // ZR):
        pltpu.sync_copy(zero_v, acc.at[pl.ds(s * TPW + k * ZR, ZR)])
    plsc.subcore_barrier()

    def body(i, _):
        base = wid * EPW + i * B
        pltpu.sync_copy(fidx_hbm.at[pl.ds(base, B)], fidx_v)
        pltpu.sync_copy(tidx_hbm.at[pl.ds(base, B)], tidx_v)
        pltpu.sync_copy(uidx_hbm.at[pl.ds(base, B)], uidx_v)
        cp_a = pltpu.async_copy(adj_hbm.at[fidx_v], w_v, sem_a)
        cp_b = pltpu.async_copy(t_hbm.at[tidx_v], rows_v, sem_b)
        cp_a.wait()
        cp_b.wait()

        def scale(g, _):
            wv = w_v[pl.ds(g * 16, 16)]
            for j in range(16):
                we = wv[j]
                e = g * 16 + j
                for k in range(D // 16):
                    rows_v[e, pl.ds(k * 16, 16)] = rows_v[e, pl.ds(k * 16, 16)] * we
            return 0
        lax.fori_loop(0, B // 16, scale, 0)
        pltpu.sync_copy(rows_v, acc.at[uidx_v], add=True)
        return 0

    lax.fori_loop(0, NB, body, 0)
    plsc.subcore_barrier()

    @pl.when(c == 0)
    def _():
        pltpu.sync_copy(acc.at[pl.ds(s * TPW, TPW)],
                        out_hbm.at[0, pl.ds(s * TPW, TPW)])

    @pl.when(c == 1)
    def _():
        pltpu.sync_copy(acc.at[pl.ds(s * TPW, TPW)],
                        out_hbm.at[1, pl.ds(s * TPW, TPW)])


def _sc_call(adj_flat, t_table, fidx, tidx, uidx):
    mesh = plsc.VectorSubcoreMesh(core_axis_name="c", subcore_axis_name="s",
                                  num_cores=NC, num_subcores=NS)
    run = pl.kernel(
        _sc_kernel,
        out_type=jax.ShapeDtypeStruct((2, ACC_ROWS, D), jnp.float32),
        mesh=mesh,
        scratch_types=[
            pltpu.VMEM((B,), jnp.int32),
            pltpu.VMEM((B,), jnp.int32),
            pltpu.VMEM((B,), jnp.int32),
            pltpu.VMEM((B,), jnp.float32),
            pltpu.VMEM((B, D), jnp.float32),
            pltpu.VMEM((ZR, D), jnp.float32),
            pltpu.VMEM_SHARED((ACC_ROWS, D), jnp.float32),
            pltpu.SemaphoreType.DMA,
            pltpu.SemaphoreType.DMA,
        ],
    )
    return run(adj_flat, t_table, fidx, tidx, uidx)


def _sum_kernel(a_ref, b_ref, out_ref):
    out_ref[...] = a_ref[0] + b_ref[0]


def _sum_partials(partials):
    bn = 400
    return pl.pallas_call(
        _sum_kernel,
        grid=(N // bn,),
        in_specs=[
            pl.BlockSpec((1, bn, D), lambda g: (0, g, 0)),
            pl.BlockSpec((1, bn, D), lambda g: (1, g, 0)),
        ],
        out_specs=pl.BlockSpec((bn, D), lambda g: (g, 0)),
        out_shape=jax.ShapeDtypeStruct((N, D), jnp.float32),
    )(partials, partials)


def kernel(adj_matrix, embeddings, edge_index, W_user, b_user, W_item, b_item):
    u = edge_index[0].astype(jnp.int32)
    v = edge_index[1].astype(jnp.int32)

    fidx = u * N + v
    vroll = jnp.roll(v, -SEP)
    tidx = vroll + jnp.where(jnp.arange(E, dtype=jnp.int32) < E - SEP, 0, N)

    pad = EPAD - E
    fidx = jnp.concatenate([fidx, jnp.zeros((pad,), jnp.int32)])
    tidx = jnp.concatenate([tidx, jnp.zeros((pad,), jnp.int32)])
    uidx = jnp.concatenate([u, jnp.full((pad,), N, jnp.int32)])

    t_table = _build_t(embeddings, W_user, b_user, W_item, b_item)
    partials = _sc_call(adj_matrix.reshape(-1), t_table, fidx, tidx, uidx)
    return _sum_partials(partials)
